# baseline (device time: 41686 ns/iter reference)
import jax
import jax.numpy as jnp
from jax import lax
from jax.experimental import pallas as pl
from jax.experimental.pallas import tpu as pltpu

N_DEV = 32
MASKS = (8, 1, 2, 4, 16)
N_R = len(MASKS)
NC = 2
N_SPLIT = 2
SLAB_OFF = (0, 256, 384, 448, 480)
SLAB_ROWS = 496


def kernel(A, B):
    m, k = A.shape
    _, n = B.shape
    colw = n // NC

    def body(a_ref, b_ref, out_ref, comm_ref, send_sems, recv_sems):
        my = lax.axis_index("i")

        barrier_sem = pltpu.get_barrier_semaphore()
        for mask in MASKS:
            pl.semaphore_signal(
                barrier_sem, inc=1,
                device_id=(my ^ mask,), device_id_type=pl.DeviceIdType.MESH,
            )
        pl.semaphore_wait(barrier_sem, N_R)

        pending = []

        def bit(mask):
            return ((my & mask) != 0).astype(jnp.int32)

        SEMS_PER_CHAIN = 16

        def _copy(c, src_off, dst_off, rows, sem_idx, mask):
            rdma = pltpu.make_async_remote_copy(
                src_ref=out_ref.at[pl.ds(src_off, rows),
                                   pl.ds(c * colw, colw)],
                dst_ref=comm_ref.at[pl.ds(dst_off, rows),
                                    pl.ds(c * colw, colw)],
                send_sem=send_sems.at[SEMS_PER_CHAIN * c + sem_idx],
                recv_sem=recv_sems.at[SEMS_PER_CHAIN * c + sem_idx],
                device_id=(my ^ mask,),
                device_id_type=pl.DeviceIdType.MESH,
            )
            rdma.start()
            pending.append(rdma)
            return rdma

        def rs_issue(c, r, off, L):
            half = L // 2
            b = bit(MASKS[r])
            keep_off = off + b * half
            send_off = off + (1 - b) * half
            if r < N_SPLIT:
                q = half // 2
                bn = bit(MASKS[r + 1])
                u = (1 - bn) * q
                lz = bn * q
                rd_u = _copy(c, send_off + u, SLAB_OFF[r] + u, q,
                             2 * r, MASKS[r])
                rd_l = _copy(c, send_off + lz, SLAB_OFF[r] + lz, q,
                             2 * r + 1, MASKS[r])
                return rd_u, rd_l, keep_off, half, u, lz, q
            rd = _copy(c, send_off, SLAB_OFF[r], half, 2 * r, MASKS[r])
            return rd, None, keep_off, half, None, None, half

        def add_slab(c, r, out_off, slab_off, rows):
            out_ref[pl.ds(out_off, rows), pl.ds(c * colw, colw)] = (
                out_ref[pl.ds(out_off, rows), pl.ds(c * colw, colw)]
                + comm_ref[pl.ds(slab_off, rows), pl.ds(c * colw, colw)]
            )

        def ag_issue(c, r, off, L):
            rdma = pltpu.make_async_remote_copy(
                src_ref=out_ref.at[pl.ds(off, L), pl.ds(c * colw, colw)],
                dst_ref=out_ref.at[pl.ds(off, L), pl.ds(c * colw, colw)],
                send_sem=send_sems.at[SEMS_PER_CHAIN * c + 9 + r],
                recv_sem=recv_sems.at[SEMS_PER_CHAIN * c + 9 + r],
                device_id=(my ^ MASKS[N_R - 1 - r],),
                device_id_type=pl.DeviceIdType.MESH,
            )
            rdma.start()
            pending.append(rdma)
            return rdma

        b0 = bit(MASKS[0])
        half0 = m // 2
        send0 = (1 - b0) * half0
        keep0 = b0 * half0
        out_ref[pl.ds(send0, half0), :] = jnp.dot(
            a_ref[pl.ds(send0, half0), :], b_ref[:, :],
            preferred_element_type=jnp.float32)
        st = {}
        for c in range(NC):
            st[c] = rs_issue(c, 0, jnp.int32(0), m)
        out_ref[pl.ds(keep0, half0), :] = jnp.dot(
            a_ref[pl.ds(keep0, half0), :], b_ref[:, :],
            preferred_element_type=jnp.float32)

        offL = {c: (jnp.int32(0), m) for c in range(NC)}
        ag_st = {}
        for r in range(N_R):
            for c in range(NC):
                rd_u, rd_l, keep_off, half, u, lz, q = st[c]
                rd_u.wait_recv()
                add_slab(c, r, keep_off + (u if rd_l is not None else 0),
                         SLAB_OFF[r] + (u if rd_l is not None else 0), q)
                offL[c] = (keep_off, half)
                if r + 1 < N_R:
                    st[c] = rs_issue(c, r + 1, keep_off, half)
                    if rd_l is not None:
                        rd_l.wait_recv()
                        add_slab(c, r, keep_off + lz, SLAB_OFF[r] + lz, q)
                else:
                    z = out_ref[pl.ds(keep_off, half), pl.ds(c * colw, colw)]
                    out_ref[pl.ds(keep_off, half), pl.ds(c * colw, colw)] = (
                        z * (1.0 / (1.0 + jnp.exp(-z))))
                    ag_st[c] = ag_issue(c, 0, keep_off, half)

        st = ag_st
        for r in range(N_R):
            mask = MASKS[N_R - 1 - r]
            b = bit(mask)
            for c in range(NC):
                st[c].wait_recv()
                o, Lc = offL[c]
                offL[c] = (o - b * Lc, 2 * Lc)
                if r + 1 < N_R:
                    st[c] = ag_issue(c, r + 1, *offL[c])

        for rdma in pending:
            rdma.wait_send()

    return pl.pallas_call(
        body,
        out_shape=jax.ShapeDtypeStruct((m, n), jnp.float32),
        in_specs=[
            pl.BlockSpec(memory_space=pltpu.VMEM),
            pl.BlockSpec(memory_space=pltpu.VMEM),
        ],
        out_specs=pl.BlockSpec(memory_space=pltpu.VMEM),
        scratch_shapes=[
            pltpu.VMEM((SLAB_ROWS, n), jnp.float32),
            pltpu.SemaphoreType.DMA((16 * NC,)),
            pltpu.SemaphoreType.DMA((16 * NC,)),
        ],
        compiler_params=pltpu.CompilerParams(collective_id=0),
    )(A, B)
